# baseline (speedup 1.0000x reference)
import functools

import jax
import jax.numpy as jnp
from jax import lax
from jax.experimental import pallas as pl
from jax.experimental.pallas import tpu as pltpu
from jax.experimental.pallas import tpu_sc as plsc

N = 10000
D = 128
G = 12
EPS = 1e-5

NC = 2
NS = 16
N_PAD = 10240
ROWS_PER_TILE = N_PAD // NS
DUMMY = 10100

E = 320000
CH = 64
K = 4
T_PER_TILE = 10240
NCH = T_PER_TILE // CH
EP = NC * NS * T_PER_TILE
EP_ARR = EP + K * CH

BR = 1024


_mesh = plsc.VectorSubcoreMesh(core_axis_name="c", subcore_axis_name="s")


@functools.partial(
    pl.kernel,
    mesh=_mesh,
    out_type=jax.ShapeDtypeStruct((NC * N_PAD, D), jnp.float32),
    scratch_types=[
        pltpu.VMEM((T_PER_TILE + K * CH,), jnp.int32),
        [pltpu.VMEM((CH,), jnp.int32)] * K,
        [pltpu.VMEM((CH, D), jnp.float32)] * K,
        pltpu.VMEM_SHARED((N_PAD, D), jnp.float32),
        [pltpu.SemaphoreType.DMA] * K,
        [pltpu.SemaphoreType.DMA] * K,
        [pltpu.SemaphoreType.DMA] * K,
    ],
)
def _sc_scatter(hh, srcp, dstp, out, idx_all, dds, rowss, acc,
                semgs, semds, semss):
    c = lax.axis_index("c")
    s = lax.axis_index("s")
    r0 = s * ROWS_PER_TILE
    tbase = (c * NS + s) * T_PER_TILE
    pltpu.sync_copy(hh.at[pl.ds(r0, ROWS_PER_TILE)],
                    acc.at[pl.ds(r0, ROWS_PER_TILE)])
    pltpu.sync_copy(srcp.at[pl.ds(tbase, T_PER_TILE + K * CH)], idx_all)
    plsc.subcore_barrier()

    def fire(i, b):
        pltpu.async_copy(hh.at[idx_all.at[pl.ds(i * CH, CH)]], rowss[b],
                         semgs[b])
        pltpu.async_copy(dstp.at[pl.ds(tbase + i * CH, CH)], dds[b], semds[b])

    def wait_g(b):
        pltpu.make_async_copy(hh.at[idx_all.at[pl.ds(0, CH)]], rowss[b],
                              semgs[b]).wait()
        pltpu.make_async_copy(dstp.at[pl.ds(tbase, CH)], dds[b],
                              semds[b]).wait()

    def wait_s(b):
        pltpu.make_async_copy(rowss[b], acc.at[dds[b]], semss[b]).wait()

    for b in range(K):
        fire(b, b)

    def group(j, carry):
        e = K * j
        for b in range(K):
            wait_g(b)
            pltpu.async_copy(rowss[b], acc.at[dds[b]], semss[b], add=True)
        for b in range(K):
            wait_s(b)
            fire(e + K + b, b)
        return carry

    lax.fori_loop(0, NCH // K, group, 0)
    for b in range(K):
        wait_g(b)
    plsc.subcore_barrier()
    pltpu.sync_copy(acc.at[pl.ds(r0, ROWS_PER_TILE)],
                    out.at[pl.ds(c * N_PAD + r0, ROWS_PER_TILE)])


@functools.partial(
    pl.kernel,
    mesh=_mesh,
    out_type=jax.ShapeDtypeStruct((NC * N_PAD, D), jnp.float32),
    scratch_types=[
        [pltpu.VMEM((CH,), jnp.int32)] * K,
        pltpu.VMEM((CH, D), jnp.float32),
        pltpu.VMEM_SHARED((N_PAD, D), jnp.float32),
        [pltpu.SemaphoreType.DMA] * K,
        [pltpu.SemaphoreType.DMA] * K,
    ],
)
def _sc_count(ones_mat, dstp, out, dds, ones_rows, acc, semds, semss):
    c = lax.axis_index("c")
    s = lax.axis_index("s")
    r0 = s * ROWS_PER_TILE
    tbase = (c * NS + s) * T_PER_TILE
    pltpu.sync_copy(ones_mat.at[pl.ds(r0, ROWS_PER_TILE)],
                    acc.at[pl.ds(r0, ROWS_PER_TILE)])
    pltpu.sync_copy(ones_mat.at[pl.ds(0, CH)], ones_rows)
    plsc.subcore_barrier()

    def fire(i, b):
        pltpu.async_copy(dstp.at[pl.ds(tbase + i * CH, CH)], dds[b], semds[b])

    def wait_d(b):
        pltpu.make_async_copy(dstp.at[pl.ds(tbase, CH)], dds[b],
                              semds[b]).wait()

    def wait_s(b):
        pltpu.make_async_copy(ones_rows, acc.at[dds[b]], semss[b]).wait()

    for b in range(K):
        fire(b, b)

    def group(j, carry):
        e = K * j
        for b in range(K):
            wait_d(b)
            pltpu.async_copy(ones_rows, acc.at[dds[b]], semss[b], add=True)
        for b in range(K):
            wait_s(b)
            fire(e + K + b, b)
        return carry

    lax.fori_loop(0, NCH // K, group, 0)
    for b in range(K):
        wait_d(b)
    plsc.subcore_barrier()
    pltpu.sync_copy(acc.at[pl.ds(r0, ROWS_PER_TILE)],
                    out.at[pl.ds(c * N_PAD + r0, ROWS_PER_TILE)])


def _row(i):
    return (i, 0)


def _rspec():
    return pl.BlockSpec((BR, D), _row)


def _rspec_hi():
    return pl.BlockSpec((BR, D), lambda i: (i + N_PAD // BR, 0))


def _cspec():
    return pl.BlockSpec((BR, 1), _row)


def _wspec():
    return pl.BlockSpec((D, D), lambda i: (0, 0))


def _bspec():
    return pl.BlockSpec((1, D), lambda i: (0, 0))


def _pre_body(x_ref, w_ref, d0_ref, d1_ref, hh_ref):
    dinv = lax.rsqrt(d0_ref[...] + d1_ref[...] - 1.0)
    hh_ref[...] = jnp.dot(x_ref[...], w_ref[...],
                          preferred_element_type=jnp.float32) * dinv


def _pre(x, w, d0, d1):
    return pl.pallas_call(
        _pre_body,
        grid=(N_PAD // BR,),
        in_specs=[_rspec(), _wspec(), _cspec(), _cspec()],
        out_specs=_rspec(),
        out_shape=jax.ShapeDtypeStruct((N_PAD, D), jnp.float32),
    )(x, w, d0, d1)


def _combine(a0_ref, a1_ref, hh_ref, bg_ref, lng_ref, lnb_ref, d0_ref, d1_ref, h_ref):
    dinv = lax.rsqrt(d0_ref[...] + d1_ref[...] - 1.0)
    z = (a0_ref[...] + a1_ref[...] - hh_ref[...]) * dinv + bg_ref[...]
    m = jnp.mean(z, axis=1, keepdims=True)
    v = jnp.mean((z - m) ** 2, axis=1, keepdims=True)
    z = (z - m) * lax.rsqrt(v + EPS) * lng_ref[...] + lnb_ref[...]
    return jnp.maximum(z, 0.0) + h_ref[...]


def _mid_body(a0_ref, a1_ref, hh_ref, h_ref, bg_ref, lng_ref, lnb_ref, w_ref,
              d0_ref, d1_ref, h_out, hh_out):
    hn = _combine(a0_ref, a1_ref, hh_ref, bg_ref, lng_ref, lnb_ref,
                  d0_ref, d1_ref, h_ref)
    dinv = lax.rsqrt(d0_ref[...] + d1_ref[...] - 1.0)
    h_out[...] = hn
    hh_out[...] = jnp.dot(hn, w_ref[...],
                          preferred_element_type=jnp.float32) * dinv


def _mid(a0, a1, hh, h, bgr, lngr, lnbr, w, d0, d1):
    return pl.pallas_call(
        _mid_body,
        grid=(N_PAD // BR,),
        in_specs=[_rspec(), _rspec_hi(), _rspec(), _rspec(), _bspec(), _bspec(),
                  _bspec(), _wspec(), _cspec(), _cspec()],
        out_specs=(_rspec(), _rspec()),
        out_shape=(jax.ShapeDtypeStruct((N_PAD, D), jnp.float32),
                   jax.ShapeDtypeStruct((N_PAD, D), jnp.float32)),
    )(a0, a1, hh, h, bgr, lngr, lnbr, w, d0, d1)


def _fin_body(a0_ref, a1_ref, hh_ref, h_ref, bg_ref, lng_ref, lnb_ref,
              go_ref, bo_ref, wcat_ref, bcat_ref, ms_ref, d0_ref, d1_ref,
              out_ref):
    hn = _combine(a0_ref, a1_ref, hh_ref, bg_ref, lng_ref, lnb_ref,
                  d0_ref, d1_ref, h_ref)
    m = jnp.mean(hn, axis=1, keepdims=True)
    v = jnp.mean((hn - m) ** 2, axis=1, keepdims=True)
    hf = (hn - m) * lax.rsqrt(v + EPS) * go_ref[...] + bo_ref[...]
    logits = jnp.dot(hf, wcat_ref[...],
                     preferred_element_type=jnp.float32) + bcat_ref[...]
    lane = lax.broadcasted_iota(jnp.int32, (BR, D), 1)
    is_pi = lane < G
    is_mu = (lane >= G) & (lane < 2 * G)
    is_sig = (lane >= 2 * G) & (lane < 3 * G)
    neg = jnp.float32(-1e30)
    masked = jnp.where(is_pi, logits, neg)
    pmax = jnp.max(masked, axis=1, keepdims=True)
    e = jnp.exp(jnp.where(is_pi, logits - pmax, neg))
    pi = e / jnp.sum(e, axis=1, keepdims=True)
    sig = jnp.exp(jnp.where(is_sig, logits, 0.0)) + ms_ref[0, 0]
    out_ref[...] = jnp.where(is_pi, pi,
                             jnp.where(is_mu, logits,
                                       jnp.where(is_sig, sig, 0.0)))


def _fin(a0, a1, hh, h, bgr, lngr, lnbr, go, bo, wcat, bcat, ms, d0, d1):
    return pl.pallas_call(
        _fin_body,
        grid=(N_PAD // BR,),
        in_specs=[_rspec(), _rspec_hi(), _rspec(), _rspec(), _bspec(), _bspec(),
                  _bspec(), _bspec(), _bspec(), _wspec(), _bspec(),
                  pl.BlockSpec((1, 1), lambda i: (0, 0)), _cspec(), _cspec()],
        out_specs=_rspec(),
        out_shape=jax.ShapeDtypeStruct((N_PAD, D), jnp.float32),
    )(a0, a1, hh, h, bgr, lngr, lnbr, go, bo, wcat, bcat, ms, d0, d1)


def kernel(x, edge_index, Wg, bg, lng, lnb, g_out, b_out, pi_W, pi_b, mu_W,
           mu_b, sigma_W, sigma_b, min_sigma):
    x_p = jnp.zeros((N_PAD, D), jnp.float32).at[:N].set(x)
    ei = edge_index.astype(jnp.int32)
    pad = EP_ARR - E
    pad_idx = N + jnp.arange(pad, dtype=jnp.int32) % (N_PAD - N)
    srcp = jnp.concatenate([ei[0], pad_idx])
    dstp = jnp.concatenate([ei[1], pad_idx])

    degs = _sc_count(jnp.ones((N_PAD, D), jnp.float32), dstp)
    d0 = degs[:N_PAD, 0:1]
    d1 = degs[N_PAD:, 0:1]

    wcat = jnp.zeros((D, D), jnp.float32)
    wcat = wcat.at[:, 0:G].set(pi_W).at[:, G:2 * G].set(mu_W)
    wcat = wcat.at[:, 2 * G:3 * G].set(sigma_W)
    bcat = jnp.zeros((1, D), jnp.float32)
    bcat = bcat.at[0, 0:G].set(pi_b).at[0, G:2 * G].set(mu_b)
    bcat = bcat.at[0, 2 * G:3 * G].set(sigma_b)
    ms = jnp.reshape(min_sigma.astype(jnp.float32), (1, 1))

    h = x_p
    hh = _pre(x_p, Wg[0], d0, d1)
    for l in range(3):
        accf = _sc_scatter(hh, srcp, dstp)
        acc = (accf, accf)
        if l < 2:
            h, hh = _mid(acc[0], acc[1], hh, h, bg[l].reshape(1, D),
                         lng[l].reshape(1, D), lnb[l].reshape(1, D),
                         Wg[l + 1], d0, d1)
        else:
            head = _fin(acc[0], acc[1], hh, h, bg[2].reshape(1, D),
                        lng[2].reshape(1, D), lnb[2].reshape(1, D),
                        g_out.reshape(1, D), b_out.reshape(1, D),
                        wcat, bcat, ms, d0, d1)
    pi = head[:N, 0:G]
    mu = head[:N, G:2 * G].reshape(N, G, 1)
    sigma = head[:N, 2 * G:3 * G].reshape(N, G, 1)
    return (pi, mu, sigma)


# speedup vs baseline: 1.0911x; 1.0911x over previous
import functools

import numpy as np

import jax
import jax.numpy as jnp
from jax import lax
from jax.experimental import pallas as pl
from jax.experimental.pallas import tpu as pltpu
from jax.experimental.pallas import tpu_sc as plsc

N = 10000
D = 128
G = 12
EPS = 1e-5

NC = 2
NS = 16
N_PAD = 10240
ROWS_PER_TILE = N_PAD // NS
DUMMY = 10100

E = 320000
CH = 128
T_PER_TILE = 10240
NCH = T_PER_TILE // CH
EP = NC * NS * T_PER_TILE
EP_ARR = EP + 2 * CH

CW = 32
BR = 1024


_mesh = plsc.VectorSubcoreMesh(core_axis_name="c", subcore_axis_name="s")


@functools.partial(
    pl.kernel,
    mesh=_mesh,
    out_type=jax.ShapeDtypeStruct((NC * N_PAD, D), jnp.float32),
    scratch_types=[
        pltpu.VMEM((T_PER_TILE + 2 * CH,), jnp.int32),
        pltpu.VMEM((CH,), jnp.int32),
        pltpu.VMEM((CH,), jnp.int32),
        pltpu.VMEM((CH, D), jnp.float32),
        pltpu.VMEM((CH, D), jnp.float32),
        pltpu.VMEM_SHARED((N_PAD, D), jnp.float32),
        pltpu.SemaphoreType.DMA,
        pltpu.SemaphoreType.DMA,
        pltpu.SemaphoreType.DMA,
        pltpu.SemaphoreType.DMA,
        pltpu.SemaphoreType.DMA,
        pltpu.SemaphoreType.DMA,
    ],
)
def _sc_scatter(hh, srcp, dstp, out, idx_all, dd0, dd1, rows0, rows1, acc,
                semg0, semg1, semd0, semd1, sems0, sems1):
    c = lax.axis_index("c")
    s = lax.axis_index("s")
    r0 = s * ROWS_PER_TILE
    tbase = (c * NS + s) * T_PER_TILE
    pltpu.sync_copy(hh.at[pl.ds(r0, ROWS_PER_TILE)],
                    acc.at[pl.ds(r0, ROWS_PER_TILE)])
    pltpu.sync_copy(srcp.at[pl.ds(tbase, T_PER_TILE + 2 * CH)], idx_all)
    plsc.subcore_barrier()

    def fire(i, rows, semg, dd, semd):
        pltpu.async_copy(hh.at[idx_all.at[pl.ds(i * CH, CH)]], rows, semg)
        pltpu.async_copy(dstp.at[pl.ds(tbase + i * CH, CH)], dd, semd)

    def wait(rows, semg, dd, semd):
        pltpu.make_async_copy(hh.at[idx_all.at[pl.ds(0, CH)]], rows,
                              semg).wait()
        pltpu.make_async_copy(dstp.at[pl.ds(tbase, CH)], dd, semd).wait()

    fire(0, rows0, semg0, dd0, semd0)
    fire(1, rows1, semg1, dd1, semd1)

    def pair(j, carry):
        e = 2 * j
        wait(rows0, semg0, dd0, semd0)
        pltpu.sync_copy(rows0, acc.at[dd0], add=True)
        fire(e + 2, rows0, semg0, dd0, semd0)
        wait(rows1, semg1, dd1, semd1)
        pltpu.sync_copy(rows1, acc.at[dd1], add=True)
        fire(e + 3, rows1, semg1, dd1, semd1)
        return carry

    lax.fori_loop(0, NCH // 2, pair, 0)
    wait(rows0, semg0, dd0, semd0)
    wait(rows1, semg1, dd1, semd1)
    plsc.subcore_barrier()
    pltpu.sync_copy(acc.at[pl.ds(r0, ROWS_PER_TILE)],
                    out.at[pl.ds(c * N_PAD + r0, ROWS_PER_TILE)])


@functools.partial(
    pl.kernel,
    mesh=_mesh,
    out_type=jax.ShapeDtypeStruct((NC * N_PAD, CW), jnp.float32),
    scratch_types=[
        pltpu.VMEM((CH,), jnp.int32),
        pltpu.VMEM((CH,), jnp.int32),
        pltpu.VMEM((CH, CW), jnp.float32),
        pltpu.VMEM_SHARED((N_PAD, CW), jnp.float32),
        pltpu.SemaphoreType.DMA,
        pltpu.SemaphoreType.DMA,
        pltpu.SemaphoreType.DMA,
        pltpu.SemaphoreType.DMA,
    ],
)
def _sc_count(ones_mat, dstp, out, dd0, dd1, ones_rows, acc, semd0, semd1,
              sems0, sems1):
    c = lax.axis_index("c")
    s = lax.axis_index("s")
    r0 = s * ROWS_PER_TILE
    tbase = (c * NS + s) * T_PER_TILE
    pltpu.sync_copy(ones_mat.at[pl.ds(r0, ROWS_PER_TILE)],
                    acc.at[pl.ds(r0, ROWS_PER_TILE)])
    pltpu.sync_copy(ones_mat.at[pl.ds(0, CH)], ones_rows)
    plsc.subcore_barrier()

    def fire(i, dd, semd):
        pltpu.async_copy(dstp.at[pl.ds(tbase + i * CH, CH)], dd, semd)

    def wait(dd, semd):
        pltpu.make_async_copy(dstp.at[pl.ds(tbase, CH)], dd, semd).wait()

    fire(0, dd0, semd0)
    fire(1, dd1, semd1)

    def pair(j, carry):
        e = 2 * j
        wait(dd0, semd0)
        pltpu.sync_copy(ones_rows, acc.at[dd0], add=True)
        fire(e + 2, dd0, semd0)
        wait(dd1, semd1)
        pltpu.sync_copy(ones_rows, acc.at[dd1], add=True)
        fire(e + 3, dd1, semd1)
        return carry

    lax.fori_loop(0, NCH // 2, pair, 0)
    wait(dd0, semd0)
    wait(dd1, semd1)
    plsc.subcore_barrier()
    pltpu.sync_copy(acc.at[pl.ds(r0, ROWS_PER_TILE)],
                    out.at[pl.ds(c * N_PAD + r0, ROWS_PER_TILE)])


def _row(i):
    return (i, 0)


def _rspec():
    return pl.BlockSpec((BR, D), _row)


def _rspec_hi():
    return pl.BlockSpec((BR, D), lambda i: (i + N_PAD // BR, 0))


def _cspec():
    return pl.BlockSpec((BR, 1), _row)


def _wspec():
    return pl.BlockSpec((D, D), lambda i: (0, 0))


def _bspec():
    return pl.BlockSpec((1, D), lambda i: (0, 0))


def _pre_body(x_ref, w_ref, d0_ref, d1_ref, hh_ref):
    dinv = lax.rsqrt(d0_ref[...] + d1_ref[...] - 1.0)
    hh_ref[...] = jnp.dot(x_ref[...], w_ref[...],
                          preferred_element_type=jnp.float32) * dinv


def _pre(x, w, d0, d1):
    return pl.pallas_call(
        _pre_body,
        grid=(N_PAD // BR,),
        in_specs=[_rspec(), _wspec(), _cspec(), _cspec()],
        out_specs=_rspec(),
        out_shape=jax.ShapeDtypeStruct((N_PAD, D), jnp.float32),
    )(x, w, d0, d1)


def _combine(a0_ref, a1_ref, hh_ref, bg_ref, lng_ref, lnb_ref, d0_ref, d1_ref, h_ref):
    dinv = lax.rsqrt(d0_ref[...] + d1_ref[...] - 1.0)
    z = (a0_ref[...] + a1_ref[...] - hh_ref[...]) * dinv + bg_ref[...]
    m = jnp.mean(z, axis=1, keepdims=True)
    v = jnp.mean((z - m) ** 2, axis=1, keepdims=True)
    z = (z - m) * lax.rsqrt(v + EPS) * lng_ref[...] + lnb_ref[...]
    return jnp.maximum(z, 0.0) + h_ref[...]


def _mid_body(a0_ref, a1_ref, hh_ref, h_ref, bg_ref, lng_ref, lnb_ref, w_ref,
              d0_ref, d1_ref, h_out, hh_out):
    hn = _combine(a0_ref, a1_ref, hh_ref, bg_ref, lng_ref, lnb_ref,
                  d0_ref, d1_ref, h_ref)
    dinv = lax.rsqrt(d0_ref[...] + d1_ref[...] - 1.0)
    h_out[...] = hn
    hh_out[...] = jnp.dot(hn, w_ref[...],
                          preferred_element_type=jnp.float32) * dinv


def _mid(a0, a1, hh, h, bgr, lngr, lnbr, w, d0, d1):
    return pl.pallas_call(
        _mid_body,
        grid=(N_PAD // BR,),
        in_specs=[_rspec(), _rspec_hi(), _rspec(), _rspec(), _bspec(), _bspec(),
                  _bspec(), _wspec(), _cspec(), _cspec()],
        out_specs=(_rspec(), _rspec()),
        out_shape=(jax.ShapeDtypeStruct((N_PAD, D), jnp.float32),
                   jax.ShapeDtypeStruct((N_PAD, D), jnp.float32)),
    )(a0, a1, hh, h, bgr, lngr, lnbr, w, d0, d1)


def _fin_body(a0_ref, a1_ref, hh_ref, h_ref, bg_ref, lng_ref, lnb_ref,
              go_ref, bo_ref, wcat_ref, bcat_ref, ms_ref, d0_ref, d1_ref,
              out_ref):
    hn = _combine(a0_ref, a1_ref, hh_ref, bg_ref, lng_ref, lnb_ref,
                  d0_ref, d1_ref, h_ref)
    m = jnp.mean(hn, axis=1, keepdims=True)
    v = jnp.mean((hn - m) ** 2, axis=1, keepdims=True)
    hf = (hn - m) * lax.rsqrt(v + EPS) * go_ref[...] + bo_ref[...]
    logits = jnp.dot(hf, wcat_ref[...],
                     preferred_element_type=jnp.float32) + bcat_ref[...]
    lane = lax.broadcasted_iota(jnp.int32, (BR, D), 1)
    is_pi = lane < G
    is_mu = (lane >= G) & (lane < 2 * G)
    is_sig = (lane >= 2 * G) & (lane < 3 * G)
    neg = jnp.float32(-1e30)
    masked = jnp.where(is_pi, logits, neg)
    pmax = jnp.max(masked, axis=1, keepdims=True)
    e = jnp.exp(jnp.where(is_pi, logits - pmax, neg))
    pi = e / jnp.sum(e, axis=1, keepdims=True)
    sig = jnp.exp(jnp.where(is_sig, logits, 0.0)) + ms_ref[0, 0]
    out_ref[...] = jnp.where(is_pi, pi,
                             jnp.where(is_mu, logits,
                                       jnp.where(is_sig, sig, 0.0)))


def _fin(a0, a1, hh, h, bgr, lngr, lnbr, go, bo, wcat, bcat, ms, d0, d1):
    return pl.pallas_call(
        _fin_body,
        grid=(N_PAD // BR,),
        in_specs=[_rspec(), _rspec_hi(), _rspec(), _rspec(), _bspec(), _bspec(),
                  _bspec(), _bspec(), _bspec(), _wspec(), _bspec(),
                  pl.BlockSpec((1, 1), lambda i: (0, 0)), _cspec(), _cspec()],
        out_specs=_rspec(),
        out_shape=jax.ShapeDtypeStruct((N_PAD, D), jnp.float32),
    )(a0, a1, hh, h, bgr, lngr, lnbr, go, bo, wcat, bcat, ms, d0, d1)


def kernel(x, edge_index, Wg, bg, lng, lnb, g_out, b_out, pi_W, pi_b, mu_W,
           mu_b, sigma_W, sigma_b, min_sigma):
    x_p = jnp.pad(x, ((0, N_PAD - N), (0, 0)))
    ei = edge_index.astype(jnp.int32)
    pad = EP_ARR - E
    pad_idx = np.asarray(N + np.arange(pad) % (N_PAD - N), dtype=np.int32)
    srcp = jnp.concatenate([ei[0], pad_idx])
    dstp = jnp.concatenate([ei[1], pad_idx])

    degs = _sc_count(jnp.ones((N_PAD, CW), jnp.float32), dstp)
    d0 = degs[:N_PAD, 0:1]
    d1 = degs[N_PAD:, 0:1]

    wcat = jnp.pad(jnp.concatenate([pi_W, mu_W, sigma_W], axis=1),
                   ((0, 0), (0, D - 3 * G)))
    bcat = jnp.pad(jnp.concatenate([pi_b, mu_b, sigma_b]),
                   (0, D - 3 * G)).reshape(1, D)
    ms = jnp.reshape(min_sigma.astype(jnp.float32), (1, 1))

    h = x_p
    hh = _pre(x_p, Wg[0], d0, d1)
    for l in range(3):
        accf = _sc_scatter(hh, srcp, dstp)
        acc = (accf, accf)
        if l < 2:
            h, hh = _mid(acc[0], acc[1], hh, h, bg[l].reshape(1, D),
                         lng[l].reshape(1, D), lnb[l].reshape(1, D),
                         Wg[l + 1], d0, d1)
        else:
            head = _fin(acc[0], acc[1], hh, h, bg[2].reshape(1, D),
                        lng[2].reshape(1, D), lnb[2].reshape(1, D),
                        g_out.reshape(1, D), b_out.reshape(1, D),
                        wcat, bcat, ms, d0, d1)
    pi = head[:N, 0:G]
    mu = head[:N, G:2 * G].reshape(N, G, 1)
    sigma = head[:N, 2 * G:3 * G].reshape(N, G, 1)
    return (pi, mu, sigma)
